# baseline (speedup 1.0000x reference)
import functools

import jax
import jax.numpy as jnp
from jax import lax
from jax.experimental import pallas as pl
from jax.experimental.pallas import tpu as pltpu
from jax.experimental.pallas import tpu_sc as plsc

B = 16384
D = 64
_EG = 0.5772156649015329
_TINY = 1.1754943508222875e-38


_NE = 100000
_CB = 4096
_NBLK = 25
_NEP = _NBLK * _CB


def _repack_body(*refs):
    n = len(refs) // 3
    for k in range(n):
        xa, xb, o = refs[2 * k], refs[2 * k + 1], refs[2 * n + k]
        x2 = jnp.concatenate([xa[...], xb[...]], axis=0)
        o[...] = jnp.transpose(x2).astype(jnp.bfloat16)


def _tc_repack(*tabs):
    n = len(tabs) // 2
    outs = pl.pallas_call(
        _repack_body,
        grid=(_NBLK,),
        in_specs=[pl.BlockSpec((D, _CB), lambda i: (0, i))] * (2 * n),
        out_specs=[pl.BlockSpec((_CB, 2 * D), lambda i: (i, 0))] * n,
        out_shape=[jax.ShapeDtypeStruct((_NEP, 2 * D), jnp.bfloat16)] * n,
    )(*[t.T for t in tabs])
    return list(outs)


_SC_NC = 2
_SC_NS = 16
_CH = 128


@functools.lru_cache(maxsize=None)
def _make_sc_gather_ent():
    mesh = plsc.VectorSubcoreMesh(core_axis_name="c", subcore_axis_name="s")

    @functools.partial(
        pl.kernel,
        mesh=mesh,
        out_type=[jax.ShapeDtypeStruct((B, 2 * D), jnp.bfloat16)] * 2,
        scratch_types=[
            pltpu.VMEM((_CH,), jnp.int32),
            pltpu.VMEM((_CH,), jnp.int32),
            pltpu.VMEM((_CH, 2 * D), jnp.bfloat16),
            pltpu.VMEM((_CH, 2 * D), jnp.bfloat16),
            pltpu.SemaphoreType.DMA,
            pltpu.SemaphoreType.DMA,
        ],
        compiler_params=pltpu.CompilerParams(use_tc_tiling_on_sc=False),
    )
    def sc_gather_ent(heads, tails, ent_pair,
                      o_h, o_t, hidx, tidx, b0, b1, gsem, wsem):
        nw = _SC_NC * _SC_NS
        n_per = B // nw
        wid = lax.axis_index("s") * _SC_NC + lax.axis_index("c")
        base = wid * n_per
        for c in range(n_per // _CH):
            co = base + c * _CH
            pltpu.sync_copy(heads.at[pl.ds(co, _CH)], hidx)
            pltpu.sync_copy(tails.at[pl.ds(co, _CH)], tidx)
            g0 = pltpu.async_copy(ent_pair.at[hidx], b0, gsem)
            g1 = pltpu.async_copy(ent_pair.at[tidx], b1, gsem)
            g0.wait()
            w0 = pltpu.async_copy(b0, o_h.at[pl.ds(co, _CH)], wsem)
            g1.wait()
            w1 = pltpu.async_copy(b1, o_t.at[pl.ds(co, _CH)], wsem)
            w0.wait()
            w1.wait()

    return sc_gather_ent


@functools.lru_cache(maxsize=None)
def _make_sc_gather_rel():
    mesh = plsc.VectorSubcoreMesh(core_axis_name="c", subcore_axis_name="s")

    @functools.partial(
        pl.kernel,
        mesh=mesh,
        out_type=[jax.ShapeDtypeStruct((B, 2 * D), jnp.bfloat16)] * 2,
        scratch_types=[
            pltpu.VMEM((_CH,), jnp.int32),
            pltpu.VMEM((_CH, 2 * D), jnp.bfloat16),
            pltpu.VMEM((_CH, 2 * D), jnp.bfloat16),
            pltpu.SemaphoreType.DMA,
            pltpu.SemaphoreType.DMA,
        ],
        compiler_params=pltpu.CompilerParams(use_tc_tiling_on_sc=False),
    )
    def sc_gather_rel(rels, rh_pair, rt_pair,
                      o_rh, o_rt, ridx, b0, b1, gsem, wsem):
        nw = _SC_NC * _SC_NS
        n_per = B // nw
        wid = lax.axis_index("s") * _SC_NC + lax.axis_index("c")
        base = wid * n_per
        for c in range(n_per // _CH):
            co = base + c * _CH
            pltpu.sync_copy(rels.at[pl.ds(co, _CH)], ridx)
            g0 = pltpu.async_copy(rh_pair.at[ridx], b0, gsem)
            g1 = pltpu.async_copy(rt_pair.at[ridx], b1, gsem)
            g0.wait()
            w0 = pltpu.async_copy(b0, o_rh.at[pl.ds(co, _CH)], wsem)
            g1.wait()
            w1 = pltpu.async_copy(b1, o_rt.at[pl.ds(co, _CH)], wsem)
            w0.wait()
            w1.wait()

    return sc_gather_rel


_TC_R = 2048


def _tc_body(he_r, te_r2, rh_r, rt_r,
             ts_r, te_r, w1_r, b1_r, w2c0_r, w2c1_r, w2c2_r, b2_r, out_r):
    he = he_r[...].astype(jnp.float32)
    tt = te_r2[...].astype(jnp.float32)
    rh = rh_r[...].astype(jnp.float32)
    rt = rt_r[...].astype(jnp.float32)

    ts1 = ts_r[...]
    h = jnp.maximum(ts1 * w1_r[...] + b1_r[...], 0.0)
    z = (b2_r[...] + h[:, 0:1] * w2c0_r[...] + h[:, 1:2] * w2c1_r[...]
         + h[:, 2:3] * w2c2_r[...])
    td = 1.0 / (1.0 + jnp.exp(-z))
    te = te_r[...]
    time = (td[:, 0:1] * te[0:1, :] + td[:, 1:2] * te[1:2, :]
            + td[:, 2:3] * te[2:3, :])

    def transform(mn, dl, tr, sc):
        trp = tr - time * jnp.sum(tr * time, axis=1, keepdims=True)
        scp = sc - time * jnp.sum(sc * time, axis=1, keepdims=True)
        mn2 = mn + trp
        dl2 = dl * scp
        return mn2, dl2, mn2 + dl2

    hmn2, hdl2, hmx2 = transform(he[:, :D], jnp.exp(he[:, D:]),
                                 rh[:, :D], rh[:, D:])
    tmn2, tdl2, tmx2 = transform(tt[:, :D], jnp.exp(tt[:, D:]),
                                 rt[:, :D], rt[:, D:])

    def lae(a, b):
        return jnp.maximum(a, b) + jnp.log1p(jnp.exp(-jnp.abs(a - b)))

    imn = jnp.maximum(lae(hmn2, tmn2), jnp.maximum(hmn2, tmn2))
    imx = jnp.minimum(-lae(-hmx2, -tmx2), jnp.minimum(hmx2, tmx2))

    c2g = 2.0 * _EG

    def log_vol(d):
        x = d - c2g
        sp = jnp.maximum(x, 0.0) + jnp.log1p(jnp.exp(-jnp.abs(x)))
        sp = jnp.maximum(sp, _TINY)
        return jnp.sum(jnp.log(sp), axis=1, keepdims=True)

    li = log_vol(imx - imn)
    lh = log_vol(hdl2)
    lt = log_vol(tdl2)
    out_r[...] = jnp.minimum(li - lh, li - lt)


def _tc_math(he, te2, rh, rt, ts, te, w1r, b1r, w2c0, w2c1, w2c2, b2r):
    grid = (B // _TC_R,)
    row = pl.BlockSpec((_TC_R, 2 * D), lambda i: (i, 0))
    one = pl.BlockSpec((_TC_R, 1), lambda i: (i, 0))
    small3 = pl.BlockSpec((1, 3), lambda i: (0, 0))
    tes = pl.BlockSpec((3, D), lambda i: (0, 0))
    return pl.pallas_call(
        _tc_body,
        grid=grid,
        in_specs=[row] * 4 + [one, tes, small3, small3, small3, small3,
                              small3, small3],
        out_specs=one,
        out_shape=jax.ShapeDtypeStruct((B, 1), jnp.float32),
    )(he, te2, rh, rt, ts, te, w1r, b1r, w2c0, w2c1, w2c2, b2r)


def kernel(samples, min_embedding, delta_embedding, time_embedding,
           W1, b1, W2, b2, rel_trans_for_head, rel_scale_for_head,
           rel_trans_for_tail, rel_scale_for_tail):
    heads = samples[:, 0]
    tails = samples[:, 1]
    rels = samples[:, 2]
    ts = samples[:, 3].astype(jnp.float32)[:, None]

    (ent_pair,) = _tc_repack(min_embedding, delta_embedding)
    g_h, g_t = _make_sc_gather_ent()(heads, tails, ent_pair)
    rh_pair, rt_pair = _tc_repack(rel_trans_for_head, rel_scale_for_head,
                                  rel_trans_for_tail, rel_scale_for_tail)
    g_rh, g_rt = _make_sc_gather_rel()(rels, rh_pair, rt_pair)

    w1r = W1.reshape(1, 3)
    b1r = b1.reshape(1, 3)
    w2c0 = W2[:, 0].reshape(1, 3)
    w2c1 = W2[:, 1].reshape(1, 3)
    w2c2 = W2[:, 2].reshape(1, 3)
    b2r = b2.reshape(1, 3)

    out = _tc_math(g_h, g_t, g_rh, g_rt, ts, time_embedding,
                   w1r, b1r, w2c0, w2c1, w2c2, b2r)
    return out[:, 0]


# speedup vs baseline: 2.2731x; 2.2731x over previous
import functools

import jax
import jax.numpy as jnp
from jax import lax
from jax.experimental import pallas as pl
from jax.experimental.pallas import tpu as pltpu
from jax.experimental.pallas import tpu_sc as plsc

B = 16384
D = 64
_EG = 0.5772156649015329
_TINY = 1.1754943508222875e-38


_NE = 100000
_CB = 4096
_NBLK = 25
_NEP = _NBLK * _CB


def _repack_body(*refs):
    n = len(refs) // 3
    for k in range(n):
        xa, xb, o = refs[2 * k], refs[2 * k + 1], refs[2 * n + k]
        x2 = jnp.concatenate([xa[...], xb[...]], axis=0)
        o[...] = jnp.transpose(x2)


def _tc_repack(*tabs):
    n = len(tabs) // 2
    outs = pl.pallas_call(
        _repack_body,
        grid=(_NBLK,),
        in_specs=[pl.BlockSpec((D, _CB), lambda i: (0, i))] * (2 * n),
        out_specs=[pl.BlockSpec((_CB, 2 * D), lambda i: (i, 0))] * n,
        out_shape=[jax.ShapeDtypeStruct((_NEP, 2 * D), jnp.float32)] * n,
    )(*[t.T for t in tabs])
    return list(outs)


_SC_NC = 2
_SC_NS = 16
_CH = 128


@functools.lru_cache(maxsize=None)
def _make_sc_gather_ent():
    mesh = plsc.VectorSubcoreMesh(core_axis_name="c", subcore_axis_name="s")

    @functools.partial(
        pl.kernel,
        mesh=mesh,
        out_type=[jax.ShapeDtypeStruct((B, 2 * D), jnp.float32)] * 2,
        scratch_types=[
            pltpu.VMEM((_CH,), jnp.int32),
            pltpu.VMEM((_CH,), jnp.int32),
            pltpu.VMEM((_CH, 2 * D), jnp.float32),
            pltpu.VMEM((_CH, 2 * D), jnp.float32),
            pltpu.SemaphoreType.DMA,
            pltpu.SemaphoreType.DMA,
        ],
        compiler_params=pltpu.CompilerParams(use_tc_tiling_on_sc=False),
    )
    def sc_gather_ent(heads, tails, ent_pair,
                      o_h, o_t, hidx, tidx, b0, b1, gsem, wsem):
        nw = _SC_NC * _SC_NS
        n_per = B // nw
        wid = lax.axis_index("s") * _SC_NC + lax.axis_index("c")
        base = wid * n_per
        for c in range(n_per // _CH):
            co = base + c * _CH
            pltpu.sync_copy(heads.at[pl.ds(co, _CH)], hidx)
            pltpu.sync_copy(tails.at[pl.ds(co, _CH)], tidx)
            g0 = pltpu.async_copy(ent_pair.at[hidx], b0, gsem)
            g1 = pltpu.async_copy(ent_pair.at[tidx], b1, gsem)
            g0.wait()
            w0 = pltpu.async_copy(b0, o_h.at[pl.ds(co, _CH)], wsem)
            g1.wait()
            w1 = pltpu.async_copy(b1, o_t.at[pl.ds(co, _CH)], wsem)
            w0.wait()
            w1.wait()

    return sc_gather_ent


@functools.lru_cache(maxsize=None)
def _make_sc_gather_rel():
    mesh = plsc.VectorSubcoreMesh(core_axis_name="c", subcore_axis_name="s")

    @functools.partial(
        pl.kernel,
        mesh=mesh,
        out_type=[jax.ShapeDtypeStruct((B, 2 * D), jnp.float32)] * 2,
        scratch_types=[
            pltpu.VMEM((_CH,), jnp.int32),
            pltpu.VMEM((_CH, 2 * D), jnp.float32),
            pltpu.VMEM((_CH, 2 * D), jnp.float32),
            pltpu.SemaphoreType.DMA,
            pltpu.SemaphoreType.DMA,
        ],
        compiler_params=pltpu.CompilerParams(use_tc_tiling_on_sc=False),
    )
    def sc_gather_rel(rels, rh_pair, rt_pair,
                      o_rh, o_rt, ridx, b0, b1, gsem, wsem):
        nw = _SC_NC * _SC_NS
        n_per = B // nw
        wid = lax.axis_index("s") * _SC_NC + lax.axis_index("c")
        base = wid * n_per
        for c in range(n_per // _CH):
            co = base + c * _CH
            pltpu.sync_copy(rels.at[pl.ds(co, _CH)], ridx)
            g0 = pltpu.async_copy(rh_pair.at[ridx], b0, gsem)
            g1 = pltpu.async_copy(rt_pair.at[ridx], b1, gsem)
            g0.wait()
            w0 = pltpu.async_copy(b0, o_rh.at[pl.ds(co, _CH)], wsem)
            g1.wait()
            w1 = pltpu.async_copy(b1, o_rt.at[pl.ds(co, _CH)], wsem)
            w0.wait()
            w1.wait()

    return sc_gather_rel


_TC_R = 4096


def _tc_body(he_r, te_r2, rh_r, rt_r,
             ts_r, te_r, w1_r, b1_r, w2c0_r, w2c1_r, w2c2_r, b2_r, out_r):
    he = he_r[...]
    tt = te_r2[...]
    rh = rh_r[...]
    rt = rt_r[...]

    ts1 = ts_r[...]
    h = jnp.maximum(ts1 * w1_r[...] + b1_r[...], 0.0)
    z = (b2_r[...] + h[:, 0:1] * w2c0_r[...] + h[:, 1:2] * w2c1_r[...]
         + h[:, 2:3] * w2c2_r[...])
    td = 1.0 / (1.0 + jnp.exp(-z))
    te = te_r[...]
    time = (td[:, 0:1] * te[0:1, :] + td[:, 1:2] * te[1:2, :]
            + td[:, 2:3] * te[2:3, :])

    def transform(mn, dl, tr, sc):
        trp = tr - time * jnp.sum(tr * time, axis=1, keepdims=True)
        scp = sc - time * jnp.sum(sc * time, axis=1, keepdims=True)
        mn2 = mn + trp
        dl2 = dl * scp
        return mn2, dl2, mn2 + dl2

    hmn2, hdl2, hmx2 = transform(he[:, :D], jnp.exp(he[:, D:]),
                                 rh[:, :D], rh[:, D:])
    tmn2, tdl2, tmx2 = transform(tt[:, :D], jnp.exp(tt[:, D:]),
                                 rt[:, :D], rt[:, D:])

    def lae(a, b):
        return jnp.maximum(a, b) + jnp.log1p(jnp.exp(-jnp.abs(a - b)))

    imn = lae(hmn2, tmn2)
    imx = -lae(-hmx2, -tmx2)

    c2g = 2.0 * _EG

    def log_vol(d):
        x = d - c2g
        sp = jnp.maximum(x, 0.0) + jnp.log1p(jnp.exp(-jnp.abs(x)))
        sp = jnp.maximum(sp, _TINY)
        return jnp.sum(jnp.log(sp), axis=1, keepdims=True)

    li = log_vol(imx - imn)
    lh = log_vol(hdl2)
    lt = log_vol(tdl2)
    out_r[...] = jnp.minimum(li - lh, li - lt)


def _tc_math(he, te2, rh, rt, ts, te, w1r, b1r, w2c0, w2c1, w2c2, b2r):
    grid = (B // _TC_R,)
    row = pl.BlockSpec((_TC_R, 2 * D), lambda i: (i, 0))
    one = pl.BlockSpec((_TC_R, 1), lambda i: (i, 0))
    small3 = pl.BlockSpec((1, 3), lambda i: (0, 0))
    tes = pl.BlockSpec((3, D), lambda i: (0, 0))
    return pl.pallas_call(
        _tc_body,
        grid=grid,
        in_specs=[row] * 4 + [one, tes, small3, small3, small3, small3,
                              small3, small3],
        out_specs=one,
        out_shape=jax.ShapeDtypeStruct((B, 1), jnp.float32),
    )(he, te2, rh, rt, ts, te, w1r, b1r, w2c0, w2c1, w2c2, b2r)


def kernel(samples, min_embedding, delta_embedding, time_embedding,
           W1, b1, W2, b2, rel_trans_for_head, rel_scale_for_head,
           rel_trans_for_tail, rel_scale_for_tail):
    heads = samples[:, 0]
    tails = samples[:, 1]
    rels = samples[:, 2]
    ts = samples[:, 3].astype(jnp.float32)[:, None]

    (ent_pair,) = _tc_repack(min_embedding, delta_embedding)
    g_h, g_t = _make_sc_gather_ent()(heads, tails, ent_pair)
    rh_pair, rt_pair = _tc_repack(rel_trans_for_head, rel_scale_for_head,
                                  rel_trans_for_tail, rel_scale_for_tail)
    g_rh, g_rt = _make_sc_gather_rel()(rels, rh_pair, rt_pair)

    w1r = W1.reshape(1, 3)
    b1r = b1.reshape(1, 3)
    w2c0 = W2[:, 0].reshape(1, 3)
    w2c1 = W2[:, 1].reshape(1, 3)
    w2c2 = W2[:, 2].reshape(1, 3)
    b2r = b2.reshape(1, 3)

    out = _tc_math(g_h, g_t, g_rh, g_rt, ts, time_embedding,
                   w1r, b1r, w2c0, w2c1, w2c2, b2r)
    return out[:, 0]
